# baseline (speedup 1.0000x reference)
import functools

import jax
import jax.numpy as jnp
from jax import lax
from jax.experimental import pallas as pl
from jax.experimental.pallas import tpu as pltpu
from jax.experimental.pallas import tpu_sc as plsc

_B, _N, _D = 16, 4096, 1280
_BN = 1024


def _maxpool_body(x_ref, pooled_ref, idx_ref):
    h = pl.program_id(1)
    xb = x_ref[0]
    bm = jnp.max(xb, axis=0)
    iota = lax.broadcasted_iota(jnp.int32, xb.shape, 0)
    bi = jnp.min(jnp.where(xb == bm[None, :], iota, _BN), axis=0) + h * _BN

    @pl.when(h == 0)
    def _init():
        pooled_ref[0, 0] = bm
        idx_ref[0, 0] = bi

    @pl.when(h > 0)
    def _merge():
        m = pooled_ref[0, 0]
        take = bm > m
        pooled_ref[0, 0] = jnp.where(take, bm, m)
        idx_ref[0, 0] = jnp.where(take, bi, idx_ref[0, 0])


def _tc_maxpool(x):
    nh = _N // _BN
    pooled, idx = pl.pallas_call(
        _maxpool_body,
        grid=(_B, nh),
        in_specs=[pl.BlockSpec((1, _BN, _D), lambda b, h: (b, h, 0))],
        out_specs=[
            pl.BlockSpec((1, 1, _D), lambda b, h: (b, 0, 0)),
            pl.BlockSpec((1, 1, _D), lambda b, h: (b, 0, 0)),
        ],
        out_shape=[
            jax.ShapeDtypeStruct((_B, 1, _D), jnp.float32),
            jax.ShapeDtypeStruct((_B, 1, _D), jnp.int32),
        ],
        compiler_params=pltpu.CompilerParams(
            dimension_semantics=("parallel", "arbitrary"),
        ),
    )(x)
    return pooled.reshape(_B, _D), idx.reshape(_B, _D)


def _sc_scatter_body(idx_hbm, out_hbm, idx_v, row_v):
    wid = lax.axis_index("s") * 2 + lax.axis_index("c")

    @pl.when(wid < _B)
    def _():
        pltpu.sync_copy(idx_hbm.at[wid], idx_v)
        zeros16 = jnp.zeros((16,), jnp.float32)

        def zbody(i, carry):
            row_v[pl.ds(i * 16, 16)] = zeros16
            return carry

        lax.fori_loop(0, _N // 16, zbody, 0)
        ones16 = jnp.ones((16,), jnp.float32)

        def sbody(i, carry):
            iv = idx_v[pl.ds(i * 16, 16)]
            plsc.store_scatter(row_v, [iv], ones16)
            return carry

        lax.fori_loop(0, _D // 16, sbody, 0)
        pltpu.sync_copy(row_v, out_hbm.at[wid])


def _sc_scatter(idx):
    call = pl.kernel(
        _sc_scatter_body,
        mesh=plsc.VectorSubcoreMesh(core_axis_name="c", subcore_axis_name="s"),
        compiler_params=pltpu.CompilerParams(needs_layout_passes=False),
        out_type=jax.ShapeDtypeStruct((_B, _N), jnp.float32),
        scratch_types=[
            pltpu.VMEM((_D,), jnp.int32),
            pltpu.VMEM((_N,), jnp.float32),
        ],
    )
    return call(idx)


def kernel(x):
    pooled, idx = _tc_maxpool(x)
    attn = _sc_scatter(idx)
    return pooled, attn


# speedup vs baseline: 1.0373x; 1.0373x over previous
import functools

import jax
import jax.numpy as jnp
from jax import lax
from jax.experimental import pallas as pl
from jax.experimental.pallas import tpu as pltpu
from jax.experimental.pallas import tpu_sc as plsc

_B, _N, _D = 16, 4096, 1280
_BN = 1024


def _maxpool_body(x_ref, pooled_ref, idx_ref):
    h = pl.program_id(1)
    xb = x_ref[0]
    bm = jnp.max(xb, axis=0)
    ridx = (_BN - lax.broadcasted_iota(jnp.int32, xb.shape, 0)).astype(jnp.float32)
    sc = jnp.max(jnp.where(xb == bm[None, :], ridx, -1.0), axis=0)
    bi = (jnp.float32(_BN) - sc).astype(jnp.int32) + h * _BN

    @pl.when(h == 0)
    def _init():
        pooled_ref[0, 0] = bm
        idx_ref[0, 0] = bi

    @pl.when(h > 0)
    def _merge():
        m = pooled_ref[0, 0]
        take = bm > m
        pooled_ref[0, 0] = jnp.where(take, bm, m)
        idx_ref[0, 0] = jnp.where(take, bi, idx_ref[0, 0])


def _tc_maxpool(x):
    nh = _N // _BN
    pooled, idx = pl.pallas_call(
        _maxpool_body,
        grid=(_B, nh),
        in_specs=[pl.BlockSpec((1, _BN, _D), lambda b, h: (b, h, 0))],
        out_specs=[
            pl.BlockSpec((1, 1, _D), lambda b, h: (b, 0, 0)),
            pl.BlockSpec((1, 1, _D), lambda b, h: (b, 0, 0)),
        ],
        out_shape=[
            jax.ShapeDtypeStruct((_B, 1, _D), jnp.float32),
            jax.ShapeDtypeStruct((_B, 1, _D), jnp.int32),
        ],
        compiler_params=pltpu.CompilerParams(
            dimension_semantics=("parallel", "arbitrary"),
        ),
    )(x)
    return pooled.reshape(_B, _D), idx.reshape(_B, _D)


def _sc_scatter_body(idx_hbm, out_hbm, idx_v, row_v):
    wid = lax.axis_index("s") * 2 + lax.axis_index("c")

    @pl.when(wid < _B)
    def _():
        pltpu.sync_copy(idx_hbm.at[wid], idx_v)
        zeros16 = jnp.zeros((16,), jnp.float32)

        def zbody(i, carry):
            row_v[pl.ds(i * 16, 16)] = zeros16
            return carry

        lax.fori_loop(0, _N // 16, zbody, 0)
        ones16 = jnp.ones((16,), jnp.float32)

        def sbody(i, carry):
            iv = idx_v[pl.ds(i * 16, 16)]
            plsc.store_scatter(row_v, [iv], ones16)
            return carry

        lax.fori_loop(0, _D // 16, sbody, 0)
        pltpu.sync_copy(row_v, out_hbm.at[wid])


def _sc_scatter(idx):
    call = pl.kernel(
        _sc_scatter_body,
        mesh=plsc.VectorSubcoreMesh(core_axis_name="c", subcore_axis_name="s"),
        compiler_params=pltpu.CompilerParams(needs_layout_passes=False),
        out_type=jax.ShapeDtypeStruct((_B, _N), jnp.float32),
        scratch_types=[
            pltpu.VMEM((_D,), jnp.int32),
            pltpu.VMEM((_N,), jnp.float32),
        ],
    )
    return call(idx)


def kernel(x):
    pooled, idx = _tc_maxpool(x)
    attn = _sc_scatter(idx)
    return pooled, attn


# speedup vs baseline: 1.1039x; 1.0641x over previous
import functools

import jax
import jax.numpy as jnp
from jax import lax
from jax.experimental import pallas as pl
from jax.experimental.pallas import tpu as pltpu
from jax.experimental.pallas import tpu_sc as plsc

_B, _N, _D = 16, 4096, 1280
_BN = 128


def _maxpool_body(x_ref, pooled_ref, idx_ref):
    j = pl.program_id(0)
    xb = x_ref[...]
    bm = jnp.max(xb, axis=1)
    ridx = (_BN - lax.broadcasted_iota(jnp.int32, xb.shape, 1)).astype(jnp.float32)
    sc = jnp.max(jnp.where(xb == bm[:, None, :], ridx, -1.0), axis=1)
    bi = (jnp.float32(_BN) - sc).astype(jnp.int32) + j * _BN

    @pl.when(j == 0)
    def _init():
        pooled_ref[...] = bm
        idx_ref[...] = bi

    @pl.when(j > 0)
    def _merge():
        m = pooled_ref[...]
        take = bm > m
        pooled_ref[...] = jnp.where(take, bm, m)
        idx_ref[...] = jnp.where(take, bi, idx_ref[...])


def _tc_maxpool(x):
    nj = _N // _BN
    return pl.pallas_call(
        _maxpool_body,
        grid=(nj,),
        in_specs=[pl.BlockSpec((_B, _BN, _D), lambda j: (0, j, 0))],
        out_specs=[
            pl.BlockSpec((_B, _D), lambda j: (0, 0)),
            pl.BlockSpec((_B, _D), lambda j: (0, 0)),
        ],
        out_shape=[
            jax.ShapeDtypeStruct((_B, _D), jnp.float32),
            jax.ShapeDtypeStruct((_B, _D), jnp.int32),
        ],
    )(x)


def _sc_scatter_body(idx_hbm, out_hbm, idx_v, row_v):
    wid = lax.axis_index("s") * 2 + lax.axis_index("c")

    @pl.when(wid < _B)
    def _():
        pltpu.sync_copy(idx_hbm.at[wid], idx_v)
        zeros16 = jnp.zeros((16,), jnp.float32)

        def zbody(i, carry):
            row_v[pl.ds(i * 16, 16)] = zeros16
            return carry

        lax.fori_loop(0, _N // 16, zbody, 0)
        ones16 = jnp.ones((16,), jnp.float32)

        def sbody(i, carry):
            iv = idx_v[pl.ds(i * 16, 16)]
            plsc.store_scatter(row_v, [iv], ones16)
            return carry

        lax.fori_loop(0, _D // 16, sbody, 0)
        pltpu.sync_copy(row_v, out_hbm.at[wid])


def _sc_scatter(idx):
    call = pl.kernel(
        _sc_scatter_body,
        mesh=plsc.VectorSubcoreMesh(core_axis_name="c", subcore_axis_name="s"),
        compiler_params=pltpu.CompilerParams(needs_layout_passes=False),
        out_type=jax.ShapeDtypeStruct((_B, _N), jnp.float32),
        scratch_types=[
            pltpu.VMEM((_D,), jnp.int32),
            pltpu.VMEM((_N,), jnp.float32),
        ],
    )
    return call(idx)


def kernel(x):
    pooled, idx = _tc_maxpool(x)
    attn = _sc_scatter(idx)
    return pooled, attn


# speedup vs baseline: 1.1435x; 1.0359x over previous
import functools

import jax
import jax.numpy as jnp
from jax import lax
from jax.experimental import pallas as pl
from jax.experimental.pallas import tpu as pltpu
from jax.experimental.pallas import tpu_sc as plsc

_B, _N, _D = 16, 4096, 1280
_BN = 256


def _maxpool_body(x_ref, pooled_ref, idx_ref):
    j = pl.program_id(0)
    xb = x_ref[...]
    bm = jnp.max(xb, axis=1)
    ridx = (_BN - lax.broadcasted_iota(jnp.int32, xb.shape, 1)).astype(jnp.float32)
    sc = jnp.max(jnp.where(xb == bm[:, None, :], ridx, -1.0), axis=1)
    bi = (jnp.float32(_BN) - sc).astype(jnp.int32) + j * _BN

    @pl.when(j == 0)
    def _init():
        pooled_ref[...] = bm
        idx_ref[...] = bi

    @pl.when(j > 0)
    def _merge():
        m = pooled_ref[...]
        take = bm > m
        pooled_ref[...] = jnp.where(take, bm, m)
        idx_ref[...] = jnp.where(take, bi, idx_ref[...])


def _tc_maxpool(x):
    nj = _N // _BN
    return pl.pallas_call(
        _maxpool_body,
        grid=(nj,),
        in_specs=[pl.BlockSpec((_B, _BN, _D), lambda j: (0, j, 0))],
        out_specs=[
            pl.BlockSpec((_B, _D), lambda j: (0, 0)),
            pl.BlockSpec((_B, _D), lambda j: (0, 0)),
        ],
        out_shape=[
            jax.ShapeDtypeStruct((_B, _D), jnp.float32),
            jax.ShapeDtypeStruct((_B, _D), jnp.int32),
        ],
    )(x)


def _sc_scatter_body(idx_hbm, out_hbm, idx_v, row_v):
    wid = lax.axis_index("s") * 2 + lax.axis_index("c")

    @pl.when(wid < _B)
    def _():
        pltpu.sync_copy(idx_hbm.at[wid], idx_v)
        zeros16 = jnp.zeros((16,), jnp.float32)

        def zbody(i, carry):
            row_v[pl.ds(i * 16, 16)] = zeros16
            return carry

        lax.fori_loop(0, _N // 16, zbody, 0)
        ones16 = jnp.ones((16,), jnp.float32)

        def sbody(i, carry):
            iv = idx_v[pl.ds(i * 16, 16)]
            plsc.store_scatter(row_v, [iv], ones16)
            return carry

        lax.fori_loop(0, _D // 16, sbody, 0)
        pltpu.sync_copy(row_v, out_hbm.at[wid])


def _sc_scatter(idx):
    call = pl.kernel(
        _sc_scatter_body,
        mesh=plsc.VectorSubcoreMesh(core_axis_name="c", subcore_axis_name="s"),
        compiler_params=pltpu.CompilerParams(needs_layout_passes=False),
        out_type=jax.ShapeDtypeStruct((_B, _N), jnp.float32),
        scratch_types=[
            pltpu.VMEM((_D,), jnp.int32),
            pltpu.VMEM((_N,), jnp.float32),
        ],
    )
    return call(idx)


def kernel(x):
    pooled, idx = _tc_maxpool(x)
    attn = _sc_scatter(idx)
    return pooled, attn


# speedup vs baseline: 1.1547x; 1.0098x over previous
import functools

import jax
import jax.numpy as jnp
from jax import lax
from jax.experimental import pallas as pl
from jax.experimental.pallas import tpu as pltpu
from jax.experimental.pallas import tpu_sc as plsc

_B, _N, _D = 16, 4096, 1280
_BN = 128
_K = 4


def _chunk_copy(x_hbm, buf, sem, j):
    k = lax.rem(j, _K)
    return pltpu.make_async_copy(
        x_hbm.at[:, pl.ds(j * _BN, _BN), :], buf.at[k], sem.at[k]
    )


def _maxpool_body(x_hbm, pooled_ref, idx_ref, buf, sem):
    nj = _N // _BN
    for j0 in range(_K):
        _chunk_copy(x_hbm, buf, sem, j0).start()

    pooled_ref[...] = jnp.full((_B, _D), -jnp.inf, jnp.float32)
    idx_ref[...] = jnp.zeros((_B, _D), jnp.int32)

    def loop_body(j, carry):
        _chunk_copy(x_hbm, buf, sem, j).wait()

        @pl.when(j + _K < nj)
        def _prefetch():
            _chunk_copy(x_hbm, buf, sem, j + _K).start()

        xb = buf[lax.rem(j, _K)]
        bm = jnp.max(xb, axis=1)
        ridx = (_BN - lax.broadcasted_iota(jnp.int32, xb.shape, 1)).astype(jnp.float32)
        sc = jnp.max(jnp.where(xb == bm[:, None, :], ridx, -1.0), axis=1)
        bi = (jnp.float32(_BN) - sc).astype(jnp.int32) + j * _BN

        m = pooled_ref[...]
        take = bm > m
        pooled_ref[...] = jnp.where(take, bm, m)
        idx_ref[...] = jnp.where(take, bi, idx_ref[...])
        return carry

    lax.fori_loop(0, nj, loop_body, 0)


def _tc_maxpool(x):
    return pl.pallas_call(
        _maxpool_body,
        in_specs=[pl.BlockSpec(memory_space=pl.ANY)],
        out_specs=[
            pl.BlockSpec(memory_space=pltpu.MemorySpace.VMEM),
            pl.BlockSpec(memory_space=pltpu.MemorySpace.VMEM),
        ],
        out_shape=[
            jax.ShapeDtypeStruct((_B, _D), jnp.float32),
            jax.ShapeDtypeStruct((_B, _D), jnp.int32),
        ],
        scratch_shapes=[
            pltpu.VMEM((_K, _B, _BN, _D), jnp.float32),
            pltpu.SemaphoreType.DMA((_K,)),
        ],
    )(x)


def _sc_scatter_body(idx_hbm, out_hbm, idx_v, row_v):
    wid = lax.axis_index("s") * 2 + lax.axis_index("c")

    @pl.when(wid < _B)
    def _():
        pltpu.sync_copy(idx_hbm.at[wid], idx_v)
        zeros16 = jnp.zeros((16,), jnp.float32)

        def zbody(i, carry):
            row_v[pl.ds(i * 16, 16)] = zeros16
            return carry

        lax.fori_loop(0, _N // 16, zbody, 0)
        ones16 = jnp.ones((16,), jnp.float32)

        def sbody(i, carry):
            iv = idx_v[pl.ds(i * 16, 16)]
            plsc.store_scatter(row_v, [iv], ones16)
            return carry

        lax.fori_loop(0, _D // 16, sbody, 0)
        pltpu.sync_copy(row_v, out_hbm.at[wid])


def _sc_scatter(idx):
    call = pl.kernel(
        _sc_scatter_body,
        mesh=plsc.VectorSubcoreMesh(core_axis_name="c", subcore_axis_name="s"),
        compiler_params=pltpu.CompilerParams(needs_layout_passes=False),
        out_type=jax.ShapeDtypeStruct((_B, _N), jnp.float32),
        scratch_types=[
            pltpu.VMEM((_D,), jnp.int32),
            pltpu.VMEM((_N,), jnp.float32),
        ],
    )
    return call(idx)


def kernel(x):
    pooled, idx = _tc_maxpool(x)
    attn = _sc_scatter(idx)
    return pooled, attn


# speedup vs baseline: 1.1579x; 1.0028x over previous
import functools

import jax
import jax.numpy as jnp
from jax import lax
from jax.experimental import pallas as pl
from jax.experimental.pallas import tpu as pltpu
from jax.experimental.pallas import tpu_sc as plsc

_B, _N, _D = 16, 4096, 1280
_BN = 128
_K = 4


def _chunk_copy(x_hbm, buf, sem, j):
    k = lax.rem(j, _K)
    return pltpu.make_async_copy(
        x_hbm.at[:, pl.ds(j * _BN, _BN), :], buf.at[k], sem.at[k]
    )


def _maxpool_body(x_hbm, pooled_ref, idx_ref, buf, sem):
    nj = _N // _BN
    for j0 in range(_K - 1):
        _chunk_copy(x_hbm, buf, sem, j0).start()

    pooled_ref[...] = jnp.full((_B, _D), -jnp.inf, jnp.float32)
    idx_ref[...] = jnp.zeros((_B, _D), jnp.int32)

    def loop_body(j, carry):
        @pl.when(j + _K - 1 < nj)
        def _prefetch():
            _chunk_copy(x_hbm, buf, sem, j + _K - 1).start()

        _chunk_copy(x_hbm, buf, sem, j).wait()

        xb = buf[lax.rem(j, _K)]
        bm = jnp.max(xb, axis=1)
        ridx = (_BN - lax.broadcasted_iota(jnp.int32, xb.shape, 1)).astype(jnp.float32)
        sc = jnp.max(jnp.where(xb == bm[:, None, :], ridx, -1.0), axis=1)
        bi = (jnp.float32(_BN) - sc).astype(jnp.int32) + j * _BN

        m = pooled_ref[...]
        take = bm > m
        pooled_ref[...] = jnp.where(take, bm, m)
        idx_ref[...] = jnp.where(take, bi, idx_ref[...])
        return carry

    lax.fori_loop(0, nj, loop_body, 0)


def _tc_maxpool(x):
    return pl.pallas_call(
        _maxpool_body,
        in_specs=[pl.BlockSpec(memory_space=pl.ANY)],
        out_specs=[
            pl.BlockSpec(memory_space=pltpu.MemorySpace.VMEM),
            pl.BlockSpec(memory_space=pltpu.MemorySpace.VMEM),
        ],
        out_shape=[
            jax.ShapeDtypeStruct((_B, _D), jnp.float32),
            jax.ShapeDtypeStruct((_B, _D), jnp.int32),
        ],
        scratch_shapes=[
            pltpu.VMEM((_K, _B, _BN, _D), jnp.float32),
            pltpu.SemaphoreType.DMA((_K,)),
        ],
    )(x)


def _sc_scatter_body(idx_hbm, out_hbm, idx_v, row_v):
    wid = lax.axis_index("s") * 2 + lax.axis_index("c")

    @pl.when(wid < _B)
    def _():
        pltpu.sync_copy(idx_hbm.at[wid], idx_v)
        zeros16 = jnp.zeros((16,), jnp.float32)

        def zbody(i, carry):
            row_v[pl.ds(i * 16, 16)] = zeros16
            return carry

        lax.fori_loop(0, _N // 16, zbody, 0)
        ones16 = jnp.ones((16,), jnp.float32)

        def sbody(i, carry):
            iv = idx_v[pl.ds(i * 16, 16)]
            plsc.store_scatter(row_v, [iv], ones16)
            return carry

        lax.fori_loop(0, _D // 16, sbody, 0)
        pltpu.sync_copy(row_v, out_hbm.at[wid])


def _sc_scatter(idx):
    call = pl.kernel(
        _sc_scatter_body,
        mesh=plsc.VectorSubcoreMesh(core_axis_name="c", subcore_axis_name="s"),
        compiler_params=pltpu.CompilerParams(needs_layout_passes=False),
        out_type=jax.ShapeDtypeStruct((_B, _N), jnp.float32),
        scratch_types=[
            pltpu.VMEM((_D,), jnp.int32),
            pltpu.VMEM((_N,), jnp.float32),
        ],
    )
    return call(idx)


def kernel(x):
    pooled, idx = _tc_maxpool(x)
    attn = _sc_scatter(idx)
    return pooled, attn
